# baseline (speedup 1.0000x reference)
import jax
import jax.numpy as jnp
from jax.experimental import pallas as pl
from jax.experimental.pallas import tpu as pltpu

_NE = 8
_D = 768
_N = 32768
_BLK = 2048
_INV_SQRT2 = 0.7071067811865476


def _router_body(x_ref, w_ref, b_ref, noise_ref,
                 vals_ref, idx_ref, loss_ref,
                 imp_ref, p_ref):
    i = pl.program_id(0)
    nb = pl.num_programs(0)

    @pl.when(i == 0)
    def _init():
        imp_ref[...] = jnp.zeros_like(imp_ref)
        p_ref[...] = jnp.zeros_like(p_ref)

    logits = jnp.dot(x_ref[...], w_ref[...],
                     preferred_element_type=jnp.float32) + b_ref[...]

    m = jnp.max(logits, axis=-1, keepdims=True)
    e = jnp.exp(logits - m)
    gates = e / jnp.sum(e, axis=-1, keepdims=True)
    imp_ref[...] += jnp.sum(gates, axis=0, keepdims=True)

    noisy = logits + noise_ref[...]
    mn = jnp.max(noisy, axis=-1, keepdims=True)
    sn = jnp.sum(jnp.exp(noisy - mn), axis=-1, keepdims=True)

    lane = jax.lax.broadcasted_iota(jnp.int32, noisy.shape, 1)
    i1 = jnp.min(jnp.where(noisy == mn, lane, _NE), axis=-1, keepdims=True)
    masked = jnp.where(lane == i1, -jnp.inf, noisy)
    m2 = jnp.max(masked, axis=-1, keepdims=True)
    i2 = jnp.min(jnp.where(masked == m2, lane, _NE), axis=-1, keepdims=True)

    gv1 = jnp.exp(mn - mn) / sn
    gv2 = jnp.exp(m2 - mn) / sn
    denom = gv1 + gv2 + 1e-20
    vals_ref[...] = jnp.concatenate([gv1 / denom, gv2 / denom], axis=-1)
    idx_ref[...] = jnp.concatenate([i1, i2], axis=-1)

    z = (m2 - logits) * _NE
    pvals = 0.5 * (1.0 - jax.lax.erf(z * _INV_SQRT2))
    p_ref[...] += jnp.sum(pvals, axis=0, keepdims=True)

    @pl.when(i == nb - 1)
    def _fin():
        imp = imp_ref[...]
        imp_mean = jnp.mean(imp)
        imp_var = jnp.sum((imp - imp_mean) ** 2) / (_NE - 1)
        imp_loss = imp_var / (imp_mean + 1e-8) ** 2
        pm = p_ref[...] / _N
        p_mean = jnp.mean(pm)
        p_var = jnp.sum((pm - p_mean) ** 2) / (_NE - 1)
        load_loss = p_var / (p_mean + 1e-8) ** 2
        loss_ref[...] = jnp.reshape(0.5 * (imp_loss + load_loss), (1, 1))


def kernel(x, W, b):
    noise = (1.0 / _NE) * jax.random.normal(
        jax.random.key(42), (_N, _NE), dtype=jnp.float32)
    b2 = b.reshape(1, _NE)

    grid = (_N // _BLK,)
    vals, idx, loss = pl.pallas_call(
        _router_body,
        grid=grid,
        in_specs=[
            pl.BlockSpec((_BLK, _D), lambda i: (i, 0)),
            pl.BlockSpec((_D, _NE), lambda i: (0, 0)),
            pl.BlockSpec((1, _NE), lambda i: (0, 0)),
            pl.BlockSpec((_BLK, _NE), lambda i: (i, 0)),
        ],
        out_specs=[
            pl.BlockSpec((_BLK, 2), lambda i: (i, 0)),
            pl.BlockSpec((_BLK, 2), lambda i: (i, 0)),
            pl.BlockSpec((1, 1), lambda i: (0, 0)),
        ],
        out_shape=[
            jax.ShapeDtypeStruct((_N, 2), jnp.float32),
            jax.ShapeDtypeStruct((_N, 2), jnp.int32),
            jax.ShapeDtypeStruct((1, 1), jnp.float32),
        ],
        scratch_shapes=[
            pltpu.VMEM((1, _NE), jnp.float32),
            pltpu.VMEM((1, _NE), jnp.float32),
        ],
        compiler_params=pltpu.CompilerParams(
            dimension_semantics=("arbitrary",),
        ),
    )(x, W, b2, noise)
    return vals, idx, loss.reshape(())


# speedup vs baseline: 1.0164x; 1.0164x over previous
import jax
import jax.numpy as jnp
from jax.experimental import pallas as pl
from jax.experimental.pallas import tpu as pltpu

_NE = 8
_D = 768
_N = 32768
_BLK = 2048
_NB = _N // _BLK
_INV_SQRT2 = 0.7071067811865476


def _router_body(x_ref, w_ref, b_ref, noise_ref,
                 vals_ref, idx_ref, imp_ref, p_ref):
    logits = jnp.dot(x_ref[...], w_ref[...],
                     preferred_element_type=jnp.float32) + b_ref[...]

    m = jnp.max(logits, axis=-1, keepdims=True)
    e = jnp.exp(logits - m)
    gates = e / jnp.sum(e, axis=-1, keepdims=True)
    imp_ref[...] = jnp.sum(gates, axis=0, keepdims=True)[None]

    noisy = logits + noise_ref[...]
    mn = jnp.max(noisy, axis=-1, keepdims=True)
    sn = jnp.sum(jnp.exp(noisy - mn), axis=-1, keepdims=True)

    lane = jax.lax.broadcasted_iota(jnp.int32, noisy.shape, 1)
    i1 = jnp.min(jnp.where(noisy == mn, lane, _NE), axis=-1, keepdims=True)
    masked = jnp.where(lane == i1, -jnp.inf, noisy)
    m2 = jnp.max(masked, axis=-1, keepdims=True)
    i2 = jnp.min(jnp.where(masked == m2, lane, _NE), axis=-1, keepdims=True)

    gv1 = 1.0 / sn
    gv2 = jnp.exp(m2 - mn) / sn
    denom = gv1 + gv2 + 1e-20
    vals_ref[...] = jnp.concatenate([gv1 / denom, gv2 / denom], axis=-1)
    idx_ref[...] = jnp.concatenate([i1, i2], axis=-1)

    z = (m2 - logits) * _NE
    pvals = 0.5 * (1.0 - jax.lax.erf(z * _INV_SQRT2))
    p_ref[...] = jnp.sum(pvals, axis=0, keepdims=True)[None]


def _loss_body(imp_ref, p_ref, loss_ref):
    imp = jnp.sum(imp_ref[...], axis=0)
    imp_mean = jnp.mean(imp)
    imp_var = jnp.sum((imp - imp_mean) ** 2) / (_NE - 1)
    imp_loss = imp_var / (imp_mean + 1e-8) ** 2
    pm = jnp.sum(p_ref[...], axis=0) / _N
    p_mean = jnp.mean(pm)
    p_var = jnp.sum((pm - p_mean) ** 2) / (_NE - 1)
    load_loss = p_var / (p_mean + 1e-8) ** 2
    loss_ref[...] = jnp.reshape(0.5 * (imp_loss + load_loss), (1, 1))


def kernel(x, W, b):
    noise = (1.0 / _NE) * jax.random.normal(
        jax.random.key(42), (_N, _NE), dtype=jnp.float32)
    b2 = b.reshape(1, _NE)

    vals, idx, imp_parts, p_parts = pl.pallas_call(
        _router_body,
        grid=(_NB,),
        in_specs=[
            pl.BlockSpec((_BLK, _D), lambda i: (i, 0)),
            pl.BlockSpec((_D, _NE), lambda i: (0, 0)),
            pl.BlockSpec((1, _NE), lambda i: (0, 0)),
            pl.BlockSpec((_BLK, _NE), lambda i: (i, 0)),
        ],
        out_specs=[
            pl.BlockSpec((_BLK, 2), lambda i: (i, 0)),
            pl.BlockSpec((_BLK, 2), lambda i: (i, 0)),
            pl.BlockSpec((1, 1, _NE), lambda i: (i, 0, 0)),
            pl.BlockSpec((1, 1, _NE), lambda i: (i, 0, 0)),
        ],
        out_shape=[
            jax.ShapeDtypeStruct((_N, 2), jnp.float32),
            jax.ShapeDtypeStruct((_N, 2), jnp.int32),
            jax.ShapeDtypeStruct((_NB, 1, _NE), jnp.float32),
            jax.ShapeDtypeStruct((_NB, 1, _NE), jnp.float32),
        ],
        compiler_params=pltpu.CompilerParams(
            dimension_semantics=("parallel",),
        ),
    )(x, W, b2, noise)

    loss = pl.pallas_call(
        _loss_body,
        out_shape=jax.ShapeDtypeStruct((1, 1), jnp.float32),
    )(imp_parts.reshape(_NB, _NE), p_parts.reshape(_NB, _NE))
    return vals, idx, loss.reshape(())


# speedup vs baseline: 2.1666x; 2.1317x over previous
import jax
import jax.numpy as jnp
from jax.experimental import pallas as pl
from jax.experimental.pallas import tpu as pltpu

_NE = 8
_D = 768
_N = 32768
_BLK = 2048
_NB = _N // _BLK
_INV_SQRT2 = 0.7071067811865476


_NOISE = (1.0 / _NE) * jax.random.normal(
    jax.random.key(42), (_N, _NE), dtype=jnp.float32)


def _router_body(x_ref, w_ref, b_ref, noise_ref,
                 vals_ref, idx_ref, imp_ref, p_ref):
    logits = jnp.dot(x_ref[...], w_ref[...],
                     preferred_element_type=jnp.float32) + b_ref[...]

    m = jnp.max(logits, axis=-1, keepdims=True)
    e = jnp.exp(logits - m)
    gates = e / jnp.sum(e, axis=-1, keepdims=True)
    imp_ref[...] = jnp.sum(gates, axis=0, keepdims=True)[None]

    noisy = logits + noise_ref[...]
    mn = jnp.max(noisy, axis=-1, keepdims=True)
    sn = jnp.sum(jnp.exp(noisy - mn), axis=-1, keepdims=True)

    lane = jax.lax.broadcasted_iota(jnp.int32, noisy.shape, 1)
    i1 = jnp.min(jnp.where(noisy == mn, lane, _NE), axis=-1, keepdims=True)
    masked = jnp.where(lane == i1, -jnp.inf, noisy)
    m2 = jnp.max(masked, axis=-1, keepdims=True)
    i2 = jnp.min(jnp.where(masked == m2, lane, _NE), axis=-1, keepdims=True)

    gv1 = 1.0 / sn
    gv2 = jnp.exp(m2 - mn) / sn
    denom = gv1 + gv2 + 1e-20
    vals_ref[...] = jnp.concatenate([gv1 / denom, gv2 / denom], axis=-1)
    idx_ref[...] = jnp.concatenate([i1, i2], axis=-1)

    z = (m2 - logits) * _NE
    pvals = 0.5 * (1.0 - jax.lax.erf(z * _INV_SQRT2))
    p_ref[...] = jnp.sum(pvals, axis=0, keepdims=True)[None]


def _loss_body(imp_ref, p_ref, loss_ref):
    imp = jnp.sum(imp_ref[...], axis=0)
    imp_mean = jnp.mean(imp)
    imp_var = jnp.sum((imp - imp_mean) ** 2) / (_NE - 1)
    imp_loss = imp_var / (imp_mean + 1e-8) ** 2
    pm = jnp.sum(p_ref[...], axis=0) / _N
    p_mean = jnp.mean(pm)
    p_var = jnp.sum((pm - p_mean) ** 2) / (_NE - 1)
    load_loss = p_var / (p_mean + 1e-8) ** 2
    loss_ref[...] = jnp.reshape(0.5 * (imp_loss + load_loss), (1, 1))


def kernel(x, W, b):
    noise = _NOISE
    b2 = b.reshape(1, _NE)

    vals, idx, imp_parts, p_parts = pl.pallas_call(
        _router_body,
        grid=(_NB,),
        in_specs=[
            pl.BlockSpec((_BLK, _D), lambda i: (i, 0)),
            pl.BlockSpec((_D, _NE), lambda i: (0, 0)),
            pl.BlockSpec((1, _NE), lambda i: (0, 0)),
            pl.BlockSpec((_BLK, _NE), lambda i: (i, 0)),
        ],
        out_specs=[
            pl.BlockSpec((_BLK, 2), lambda i: (i, 0)),
            pl.BlockSpec((_BLK, 2), lambda i: (i, 0)),
            pl.BlockSpec((1, 1, _NE), lambda i: (i, 0, 0)),
            pl.BlockSpec((1, 1, _NE), lambda i: (i, 0, 0)),
        ],
        out_shape=[
            jax.ShapeDtypeStruct((_N, 2), jnp.float32),
            jax.ShapeDtypeStruct((_N, 2), jnp.int32),
            jax.ShapeDtypeStruct((_NB, 1, _NE), jnp.float32),
            jax.ShapeDtypeStruct((_NB, 1, _NE), jnp.float32),
        ],
        compiler_params=pltpu.CompilerParams(
            dimension_semantics=("parallel",),
        ),
    )(x, W, b2, noise)

    loss = pl.pallas_call(
        _loss_body,
        out_shape=jax.ShapeDtypeStruct((1, 1), jnp.float32),
    )(imp_parts.reshape(_NB, _NE), p_parts.reshape(_NB, _NE))
    return vals, idx, loss.reshape(())


# speedup vs baseline: 2.5338x; 1.1695x over previous
import jax
import jax.numpy as jnp
from jax.experimental import pallas as pl
from jax.experimental.pallas import tpu as pltpu

_NE = 8
_D = 768
_N = 32768
_BLK = 2048
_NB = _N // _BLK
_INV_SQRT2 = 0.7071067811865476

def _make_noise_t():
    return jnp.transpose(
        (1.0 / _NE) * jax.random.normal(
            jax.random.key(42), (_N, _NE), dtype=jnp.float32))


try:
    _NOISE_T = _make_noise_t()
except Exception:
    _NOISE_T = None


def _router_body(x_ref, w_ref, b_ref, noise_ref,
                 vals_ref, idx_ref, imp_ref, p_ref):
    logits_tok = jnp.dot(x_ref[...], w_ref[...],
                         preferred_element_type=jnp.float32)
    lt = jnp.transpose(logits_tok) + b_ref[...]

    m = jnp.max(lt, axis=0, keepdims=True)
    e = jnp.exp(lt - m)
    gates = e / jnp.sum(e, axis=0, keepdims=True)
    imp_ref[...] = jnp.sum(gates, axis=1, keepdims=True)[None]

    noisy = lt + noise_ref[...]
    mn = jnp.max(noisy, axis=0, keepdims=True)
    sn = jnp.sum(jnp.exp(noisy - mn), axis=0, keepdims=True)

    sub = jax.lax.broadcasted_iota(jnp.int32, noisy.shape, 0)
    i1 = jnp.min(jnp.where(noisy == mn, sub, _NE), axis=0, keepdims=True)
    masked = jnp.where(sub == i1, -jnp.inf, noisy)
    m2 = jnp.max(masked, axis=0, keepdims=True)
    i2 = jnp.min(jnp.where(masked == m2, sub, _NE), axis=0, keepdims=True)

    gv1 = 1.0 / sn
    gv2 = jnp.exp(m2 - mn) / sn
    denom = gv1 + gv2 + 1e-20
    vals_ref[...] = jnp.transpose(
        jnp.concatenate([gv1 / denom, gv2 / denom], axis=0))
    idx_ref[...] = jnp.transpose(jnp.concatenate([i1, i2], axis=0))

    z = (m2 - lt) * _NE
    pvals = 0.5 * (1.0 - jax.lax.erf(z * _INV_SQRT2))
    p_ref[...] = jnp.sum(pvals, axis=1, keepdims=True)[None]


def _loss_body(imp_ref, p_ref, loss_ref):
    imp = jnp.sum(imp_ref[...], axis=0)
    imp_mean = jnp.mean(imp)
    imp_var = jnp.sum((imp - imp_mean) ** 2) / (_NE - 1)
    imp_loss = imp_var / (imp_mean + 1e-8) ** 2
    pm = jnp.sum(p_ref[...], axis=0) / _N
    p_mean = jnp.mean(pm)
    p_var = jnp.sum((pm - p_mean) ** 2) / (_NE - 1)
    load_loss = p_var / (p_mean + 1e-8) ** 2
    loss_ref[...] = jnp.reshape(0.5 * (imp_loss + load_loss), (1, 1))


def kernel(x, W, b):
    b_t = b.reshape(_NE, 1)
    noise_t = _NOISE_T if _NOISE_T is not None else _make_noise_t()

    vals, idx, imp_parts, p_parts = pl.pallas_call(
        _router_body,
        grid=(_NB,),
        in_specs=[
            pl.BlockSpec((_BLK, _D), lambda i: (i, 0)),
            pl.BlockSpec((_D, _NE), lambda i: (0, 0)),
            pl.BlockSpec((_NE, 1), lambda i: (0, 0)),
            pl.BlockSpec((_NE, _BLK), lambda i: (0, i)),
        ],
        out_specs=[
            pl.BlockSpec((_BLK, 2), lambda i: (i, 0)),
            pl.BlockSpec((_BLK, 2), lambda i: (i, 0)),
            pl.BlockSpec((1, _NE, 1), lambda i: (i, 0, 0)),
            pl.BlockSpec((1, _NE, 1), lambda i: (i, 0, 0)),
        ],
        out_shape=[
            jax.ShapeDtypeStruct((_N, 2), jnp.float32),
            jax.ShapeDtypeStruct((_N, 2), jnp.int32),
            jax.ShapeDtypeStruct((_NB, _NE, 1), jnp.float32),
            jax.ShapeDtypeStruct((_NB, _NE, 1), jnp.float32),
        ],
        compiler_params=pltpu.CompilerParams(
            dimension_semantics=("parallel",),
        ),
    )(x, W, b_t, noise_t)

    loss = pl.pallas_call(
        _loss_body,
        out_shape=jax.ShapeDtypeStruct((1, 1), jnp.float32),
    )(imp_parts.reshape(_NB, _NE), p_parts.reshape(_NB, _NE))
    return vals, idx, loss.reshape(())


# speedup vs baseline: 4.2857x; 1.6914x over previous
import jax
import jax.numpy as jnp
from jax.experimental import pallas as pl
from jax.experimental.pallas import tpu as pltpu

_NE = 8
_D = 768
_N = 32768
_BLK = 2048
_NB = _N // _BLK
_INV_SQRT2 = 0.7071067811865476

def _make_noise_t():
    return jnp.transpose(
        (1.0 / _NE) * jax.random.normal(
            jax.random.key(42), (_N, _NE), dtype=jnp.float32))


try:
    _NOISE_T = _make_noise_t()
except Exception:
    _NOISE_T = None


def _router_body(x_ref, w_ref, b_ref, noise_ref,
                 vals_ref, idx_ref, imp_ref, p_ref):
    logits_tok = jnp.dot(x_ref[...], w_ref[...],
                         preferred_element_type=jnp.float32)
    lt = jnp.transpose(logits_tok) + b_ref[...]

    m = jnp.max(lt, axis=0, keepdims=True)
    e = jnp.exp(lt - m)
    gates = e / jnp.sum(e, axis=0, keepdims=True)
    imp_ref[...] = jnp.sum(gates, axis=1, keepdims=True)[None]

    noisy = lt + noise_ref[...]
    mn = jnp.max(noisy, axis=0, keepdims=True)
    sn = jnp.sum(jnp.exp(noisy - mn), axis=0, keepdims=True)

    sub = jax.lax.broadcasted_iota(jnp.int32, noisy.shape, 0)
    i1 = jnp.min(jnp.where(noisy == mn, sub, _NE), axis=0, keepdims=True)
    masked = jnp.where(sub == i1, -jnp.inf, noisy)
    m2 = jnp.max(masked, axis=0, keepdims=True)
    i2 = jnp.min(jnp.where(masked == m2, sub, _NE), axis=0, keepdims=True)

    gv1 = 1.0 / sn
    gv2 = jnp.exp(m2 - mn) / sn
    denom = gv1 + gv2 + 1e-20
    vals_ref[...] = jnp.concatenate([gv1 / denom, gv2 / denom], axis=0)
    idx_ref[...] = jnp.concatenate([i1, i2], axis=0)

    z = (m2 - lt) * _NE
    pvals = 0.5 * (1.0 - jax.lax.erf(z * _INV_SQRT2))
    p_ref[...] = jnp.sum(pvals, axis=1, keepdims=True)[None]


def _loss_body(imp_ref, p_ref, loss_ref):
    imp = jnp.sum(imp_ref[...], axis=0)
    imp_mean = jnp.mean(imp)
    imp_var = jnp.sum((imp - imp_mean) ** 2) / (_NE - 1)
    imp_loss = imp_var / (imp_mean + 1e-8) ** 2
    pm = jnp.sum(p_ref[...], axis=0) / _N
    p_mean = jnp.mean(pm)
    p_var = jnp.sum((pm - p_mean) ** 2) / (_NE - 1)
    load_loss = p_var / (p_mean + 1e-8) ** 2
    loss_ref[...] = jnp.reshape(0.5 * (imp_loss + load_loss), (1, 1))


def kernel(x, W, b):
    b_t = b.reshape(_NE, 1)
    noise_t = _NOISE_T if _NOISE_T is not None else _make_noise_t()

    vals, idx, imp_parts, p_parts = pl.pallas_call(
        _router_body,
        grid=(_NB,),
        in_specs=[
            pl.BlockSpec((_BLK, _D), lambda i: (i, 0)),
            pl.BlockSpec((_D, _NE), lambda i: (0, 0)),
            pl.BlockSpec((_NE, 1), lambda i: (0, 0)),
            pl.BlockSpec((_NE, _BLK), lambda i: (0, i)),
        ],
        out_specs=[
            pl.BlockSpec((2, _BLK), lambda i: (0, i)),
            pl.BlockSpec((2, _BLK), lambda i: (0, i)),
            pl.BlockSpec((1, _NE, 1), lambda i: (i, 0, 0)),
            pl.BlockSpec((1, _NE, 1), lambda i: (i, 0, 0)),
        ],
        out_shape=[
            jax.ShapeDtypeStruct((2, _N), jnp.float32),
            jax.ShapeDtypeStruct((2, _N), jnp.int32),
            jax.ShapeDtypeStruct((_NB, _NE, 1), jnp.float32),
            jax.ShapeDtypeStruct((_NB, _NE, 1), jnp.float32),
        ],
        compiler_params=pltpu.CompilerParams(
            dimension_semantics=("parallel",),
        ),
    )(x, W, b_t, noise_t)

    loss = pl.pallas_call(
        _loss_body,
        out_shape=jax.ShapeDtypeStruct((1, 1), jnp.float32),
    )(imp_parts.reshape(_NB, _NE), p_parts.reshape(_NB, _NE))
    return jnp.transpose(vals), jnp.transpose(idx), loss.reshape(())


# speedup vs baseline: 4.6631x; 1.0881x over previous
import jax
import jax.numpy as jnp
from jax.experimental import pallas as pl
from jax.experimental.pallas import tpu as pltpu

_NE = 8
_D = 768
_N = 32768
_BLK = 2048
_NB = _N // _BLK
_INV_SQRT2 = 0.7071067811865476


def _make_noise_t():
    return jnp.transpose(
        (1.0 / _NE) * jax.random.normal(
            jax.random.key(42), (_N, _NE), dtype=jnp.float32))


try:
    _NOISE_T = _make_noise_t()
except Exception:
    _NOISE_T = None


def _router_body(x_ref, w_ref, b_ref, noise_ref,
                 vals_ref, idx_ref, loss_ref, imp_ref, p_ref):
    i = pl.program_id(0)

    @pl.when(i == 0)
    def _init():
        imp_ref[...] = jnp.zeros_like(imp_ref)
        p_ref[...] = jnp.zeros_like(p_ref)

    logits_tok = jnp.dot(x_ref[...], w_ref[...],
                         preferred_element_type=jnp.float32)
    lt = jnp.transpose(logits_tok) + b_ref[...]

    m = jnp.max(lt, axis=0, keepdims=True)
    e = jnp.exp(lt - m)
    gates = e / jnp.sum(e, axis=0, keepdims=True)
    imp_ref[...] += jnp.sum(gates, axis=1, keepdims=True)

    noisy = lt + noise_ref[...]
    mn = jnp.max(noisy, axis=0, keepdims=True)
    sn = jnp.sum(jnp.exp(noisy - mn), axis=0, keepdims=True)

    sub = jax.lax.broadcasted_iota(jnp.int32, noisy.shape, 0)
    i1 = jnp.min(jnp.where(noisy == mn, sub, _NE), axis=0, keepdims=True)
    masked = jnp.where(sub == i1, -jnp.inf, noisy)
    m2 = jnp.max(masked, axis=0, keepdims=True)
    i2 = jnp.min(jnp.where(masked == m2, sub, _NE), axis=0, keepdims=True)

    gv1 = 1.0 / sn
    gv2 = jnp.exp(m2 - mn) / sn
    denom = gv1 + gv2 + 1e-20
    vals_ref[...] = jnp.concatenate([gv1 / denom, gv2 / denom], axis=0)
    idx_ref[...] = jnp.concatenate([i1, i2], axis=0)

    z = (m2 - lt) * _NE
    pvals = 0.5 * (1.0 - jax.lax.erf(z * _INV_SQRT2))
    p_ref[...] += jnp.sum(pvals, axis=1, keepdims=True)

    @pl.when(i == _NB - 1)
    def _fin():
        imp = imp_ref[...]
        imp_mean = jnp.mean(imp)
        imp_var = jnp.sum((imp - imp_mean) ** 2) / (_NE - 1)
        imp_loss = imp_var / (imp_mean + 1e-8) ** 2
        pm = p_ref[...] / _N
        p_mean = jnp.mean(pm)
        p_var = jnp.sum((pm - p_mean) ** 2) / (_NE - 1)
        load_loss = p_var / (p_mean + 1e-8) ** 2
        loss_ref[...] = jnp.reshape(0.5 * (imp_loss + load_loss), (1, 1))


def kernel(x, W, b):
    b_t = b.reshape(_NE, 1)
    noise_t = _NOISE_T if _NOISE_T is not None else _make_noise_t()

    vals, idx, loss = pl.pallas_call(
        _router_body,
        grid=(_NB,),
        in_specs=[
            pl.BlockSpec((_BLK, _D), lambda i: (i, 0)),
            pl.BlockSpec((_D, _NE), lambda i: (0, 0)),
            pl.BlockSpec((_NE, 1), lambda i: (0, 0)),
            pl.BlockSpec((_NE, _BLK), lambda i: (0, i)),
        ],
        out_specs=[
            pl.BlockSpec((2, _BLK), lambda i: (0, i)),
            pl.BlockSpec((2, _BLK), lambda i: (0, i)),
            pl.BlockSpec((1, 1), lambda i: (0, 0)),
        ],
        out_shape=[
            jax.ShapeDtypeStruct((2, _N), jnp.float32),
            jax.ShapeDtypeStruct((2, _N), jnp.int32),
            jax.ShapeDtypeStruct((1, 1), jnp.float32),
        ],
        scratch_shapes=[
            pltpu.VMEM((_NE, 1), jnp.float32),
            pltpu.VMEM((_NE, 1), jnp.float32),
        ],
        compiler_params=pltpu.CompilerParams(
            dimension_semantics=("arbitrary",),
        ),
    )(x, W, b_t, noise_t)
    return jnp.transpose(vals), jnp.transpose(idx), loss.reshape(())


# speedup vs baseline: 4.8312x; 1.0360x over previous
import jax
import jax.numpy as jnp
from jax.experimental import pallas as pl
from jax.experimental.pallas import tpu as pltpu

_NE = 8
_D = 768
_N = 32768
_BLK = 4096
_NB = _N // _BLK
_INV_SQRT2 = 0.7071067811865476


def _make_noise_t():
    return jnp.transpose(
        (1.0 / _NE) * jax.random.normal(
            jax.random.key(42), (_N, _NE), dtype=jnp.float32))


try:
    _NOISE_T = _make_noise_t()
except Exception:
    _NOISE_T = None


def _router_body(x_ref, w_ref, b_ref, noise_ref,
                 vals_ref, idx_ref, loss_ref, imp_ref, p_ref):
    i = pl.program_id(0)

    @pl.when(i == 0)
    def _init():
        imp_ref[...] = jnp.zeros_like(imp_ref)
        p_ref[...] = jnp.zeros_like(p_ref)

    logits_tok = jnp.dot(x_ref[...], w_ref[...],
                         preferred_element_type=jnp.float32)
    lt = jnp.transpose(logits_tok) + b_ref[...]

    m = jnp.max(lt, axis=0, keepdims=True)
    e = jnp.exp(lt - m)
    gates = e / jnp.sum(e, axis=0, keepdims=True)
    imp_ref[...] += jnp.sum(gates, axis=1, keepdims=True)

    noisy = lt + noise_ref[...]
    mn = jnp.max(noisy, axis=0, keepdims=True)
    sn = jnp.sum(jnp.exp(noisy - mn), axis=0, keepdims=True)

    sub = jax.lax.broadcasted_iota(jnp.int32, noisy.shape, 0)
    i1 = jnp.min(jnp.where(noisy == mn, sub, _NE), axis=0, keepdims=True)
    masked = jnp.where(sub == i1, -jnp.inf, noisy)
    m2 = jnp.max(masked, axis=0, keepdims=True)
    i2 = jnp.min(jnp.where(masked == m2, sub, _NE), axis=0, keepdims=True)

    gv1 = 1.0 / sn
    gv2 = jnp.exp(m2 - mn) / sn
    denom = gv1 + gv2 + 1e-20
    vals_ref[...] = jnp.concatenate([gv1 / denom, gv2 / denom], axis=0)
    idx_ref[...] = jnp.concatenate([i1, i2], axis=0)

    z = (m2 - lt) * _NE
    pvals = 0.5 * (1.0 - jax.lax.erf(z * _INV_SQRT2))
    p_ref[...] += jnp.sum(pvals, axis=1, keepdims=True)

    @pl.when(i == _NB - 1)
    def _fin():
        imp = imp_ref[...]
        imp_mean = jnp.mean(imp)
        imp_var = jnp.sum((imp - imp_mean) ** 2) / (_NE - 1)
        imp_loss = imp_var / (imp_mean + 1e-8) ** 2
        pm = p_ref[...] / _N
        p_mean = jnp.mean(pm)
        p_var = jnp.sum((pm - p_mean) ** 2) / (_NE - 1)
        load_loss = p_var / (p_mean + 1e-8) ** 2
        loss_ref[...] = jnp.reshape(0.5 * (imp_loss + load_loss), (1, 1))


def kernel(x, W, b):
    b_t = b.reshape(_NE, 1)
    noise_t = _NOISE_T if _NOISE_T is not None else _make_noise_t()

    vals, idx, loss = pl.pallas_call(
        _router_body,
        grid=(_NB,),
        in_specs=[
            pl.BlockSpec((_BLK, _D), lambda i: (i, 0)),
            pl.BlockSpec((_D, _NE), lambda i: (0, 0)),
            pl.BlockSpec((_NE, 1), lambda i: (0, 0)),
            pl.BlockSpec((_NE, _BLK), lambda i: (0, i)),
        ],
        out_specs=[
            pl.BlockSpec((2, _BLK), lambda i: (0, i)),
            pl.BlockSpec((2, _BLK), lambda i: (0, i)),
            pl.BlockSpec((1, 1), lambda i: (0, 0)),
        ],
        out_shape=[
            jax.ShapeDtypeStruct((2, _N), jnp.float32),
            jax.ShapeDtypeStruct((2, _N), jnp.int32),
            jax.ShapeDtypeStruct((1, 1), jnp.float32),
        ],
        scratch_shapes=[
            pltpu.VMEM((_NE, 1), jnp.float32),
            pltpu.VMEM((_NE, 1), jnp.float32),
        ],
        compiler_params=pltpu.CompilerParams(
            dimension_semantics=("arbitrary",),
        ),
    )(x, W, b_t, noise_t)
    return jnp.transpose(vals), jnp.transpose(idx), loss.reshape(())
